# baseline (speedup 1.0000x reference)
import dataclasses
import functools

import jax
import jax.numpy as jnp
from jax import lax
from jax.experimental import pallas as pl
from jax.experimental.pallas import tpu as pltpu
from jax.experimental.pallas import tpu_sc as plsc

L = 16
NW = 32
N_TOK = 32768
C_DIM = 768
N_EXP = 8
N_SC = 12288
ROWS_PER_WORKER = N_SC // NW
XBLK = 64
NBLK = ROWS_PER_WORKER // XBLK
NCH = C_DIM // L
R_INFLIGHT = 4
TC_BLK = 4096


def _vgather(v, idx):
    return v.at[idx].get(mode="promise_in_bounds")


def _hsum16(v, iota):
    for sh in (8, 4, 2, 1):
        v = v + _vgather(v, jnp.bitwise_xor(iota, sh))
    return v


def _rsqrt16(x):
    i = lax.bitcast_convert_type(x, jnp.int32)
    i = jnp.int32(0x5F3759DF) - lax.shift_right_logical(i, 1)
    y = lax.bitcast_convert_type(i, jnp.float32)
    for _ in range(3):
        y = y * (jnp.float32(1.5) - jnp.float32(0.5) * x * y * y)
    return y


def _bf16_rne(x):
    u = lax.bitcast_convert_type(x, jnp.uint32)
    half = jnp.uint32(0x7FFF) + (
        lax.shift_right_logical(u, jnp.uint32(16)) & jnp.uint32(1))
    r = (u + half) & jnp.uint32(0xFFFF0000)
    return lax.bitcast_convert_type(r, jnp.float32)


def _sc_gating(x_hbm, w_hbm, aux_hbm, rw_hbm, lg_hbm, am_hbm,
               xbuf, wbuf, wt, auxv, rwst, lgst, amst, xsem):
    cid = lax.axis_index("c")
    sid = lax.axis_index("s")
    wid = sid * 2 + cid
    worker_base = wid * ROWS_PER_WORKER

    iota = lax.iota(jnp.int32, L)

    pltpu.sync_copy(w_hbm, wbuf)
    pltpu.sync_copy(aux_hbm, auxv)

    wsqv = jnp.zeros((L,), jnp.float32)
    for e in range(N_EXP):

        def _wch(i, acc, e=e):
            idx = (i * L + iota) * N_EXP + e
            v = plsc.load_gather(wbuf, [idx])
            return acc + v * v

        acc = lax.fori_loop(0, NCH, _wch, jnp.zeros((L,), jnp.float32))
        wsqv = jnp.where(iota == e, _hsum16(acc, iota), wsqv)

    winv = _rsqrt16(jnp.maximum(wsqv, jnp.float32(1e-24)))
    avec = auxv[pl.ds(0, L)]
    g_bc = [_vgather(avec, jnp.full((L,), e, jnp.int32)) for e in range(N_EXP)]
    k_bc = _vgather(avec, jnp.full((L,), N_EXP, jnp.int32))

    for e in range(N_EXP):
        wibc = _vgather(winv, jnp.full((L,), e, jnp.int32))

        def _wch2(i, _, e=e, wibc=wibc):
            idx = (i * L + iota) * N_EXP + e
            v = plsc.load_gather(wbuf, [idx])
            wt[e, pl.ds(i * L, L)] = _bf16_rne(v * wibc)
            return 0

        lax.fori_loop(0, NCH, _wch2, 0)

    def _start(blk, bb):
        rb = worker_base + blk * XBLK
        pltpu.async_copy(x_hbm.at[pl.ds(rb, XBLK), :], xbuf.at[bb],
                         xsem.at[bb])

    def _wait(bb):
        pltpu.make_async_copy(x_hbm.at[pl.ds(0, XBLK), :], xbuf.at[bb],
                              xsem.at[bb]).wait()

    def _compute(blk, bb):
        rowbase = worker_base + blk * XBLK
        xb = xbuf.at[bb]

        @pl.loop(0, XBLK // L)
        def _group(g):
            row_g = g * L
            z = jnp.zeros((L,), jnp.float32)

            ssv = z
            for q in range(L // R_INFLIGHT):
                row0 = row_g + q * R_INFLIGHT

                def _ss(i, sss, row0=row0):
                    off = i * L
                    out = []
                    for r in range(R_INFLIGHT):
                        xv = xb[row0 + r, pl.ds(off, L)]
                        out.append(sss[r] + xv * xv)
                    return tuple(out)

                sss = lax.fori_loop(0, NCH, _ss,
                                    tuple(z for _ in range(R_INFLIGHT)))
                for r in range(R_INFLIGHT):
                    mcol = iota == (q * R_INFLIGHT + r)
                    ssv = jnp.where(mcol, _hsum16(sss[r], iota), ssv)

            xinv = _rsqrt16(jnp.maximum(ssv, jnp.float32(1e-24)))

            dvs = [z for _ in range(N_EXP)]
            for q in range(L // R_INFLIGHT):
                row0 = row_g + q * R_INFLIGHT
                xibc = [_vgather(xinv,
                                 jnp.full((L,), q * R_INFLIGHT + r, jnp.int32))
                        for r in range(R_INFLIGHT)]

                def _ch(i, accs, row0=row0, xibc=xibc):
                    off = i * L
                    wv = [wt[e, pl.ds(off, L)] for e in range(N_EXP)]
                    naccs = []
                    for r in range(R_INFLIGHT):
                        xv = xb[row0 + r, pl.ds(off, L)]
                        xnr = _bf16_rne(xv * xibc[r])
                        naccs.append(tuple(accs[r][e] + xnr * wv[e]
                                           for e in range(N_EXP)))
                    return tuple(naccs)

                accs0 = tuple(tuple(z for _ in range(N_EXP))
                              for _ in range(R_INFLIGHT))
                accs = lax.fori_loop(0, NCH, _ch, accs0)
                for r in range(R_INFLIGHT):
                    mcol = iota == (q * R_INFLIGHT + r)
                    for e in range(N_EXP):
                        dvs[e] = jnp.where(mcol, _hsum16(accs[r][e], iota),
                                           dvs[e])

            logits = [dvs[e] - g_bc[e] for e in range(N_EXP)]

            act = [logits[e] > jnp.float32(0.0) for e in range(N_EXP)]
            anyact = act[0]
            for e in range(1, N_EXP):
                anyact = jnp.logical_or(anyact, act[e])
            inactive = jnp.logical_not(anyact)

            one = jnp.full((L,), 1.0, jnp.float32)
            zero = jnp.full((L,), 0.0, jnp.float32)
            rank = [zero for _ in range(N_EXP)]
            for a in range(N_EXP):
                for b in range(a + 1, N_EXP):
                    t = jnp.where(logits[a] >= logits[b], one, zero)
                    rank[b] = rank[b] + t
                    rank[a] = rank[a] + (one - t)
            fb = [rank[e] < k_bc for e in range(N_EXP)]

            mask = [jnp.where(inactive, fb[e], act[e]) for e in range(N_EXP)]
            maskf = [jnp.where(mask[e], 1.0, 0.0) for e in range(N_EXP)]
            gated = [jnp.maximum(logits[e], 0.0) for e in range(N_EXP)]
            neg = jnp.float32(-1e30)
            gm = [jnp.where(mask[e], gated[e], neg) for e in range(N_EXP)]
            mx = gm[0]
            for e in range(1, N_EXP):
                mx = jnp.maximum(mx, gm[e])
            ex = [jnp.exp(gm[e] - mx) for e in range(N_EXP)]
            s = ex[0]
            for e in range(1, N_EXP):
                s = s + ex[e]
            rw = [ex[e] / s for e in range(N_EXP)]

            ridx8 = (row_g + iota) * N_EXP
            for e in range(N_EXP):
                idx = ridx8 + e
                plsc.store_scatter(rwst, [idx], rw[e])
                plsc.store_scatter(lgst, [idx], logits[e])
                plsc.store_scatter(amst, [idx], maskf[e])

        outbase = rowbase * N_EXP
        pltpu.sync_copy(rwst, rw_hbm.at[pl.ds(outbase, XBLK * N_EXP)])
        pltpu.sync_copy(lgst, lg_hbm.at[pl.ds(outbase, XBLK * N_EXP)])
        pltpu.sync_copy(amst, am_hbm.at[pl.ds(outbase, XBLK * N_EXP)])

    _start(0, 0)

    @pl.loop(0, NBLK // 2)
    def _pair(p):
        blk0 = p * 2
        _start(blk0 + 1, 1)
        _wait(0)
        _compute(blk0, 0)
        _start(blk0 + 2, 0)
        _wait(1)
        _compute(blk0 + 1, 1)

    _wait(0)


def _tc_gating_block(x_ref, w_ref, g_ref, k_ref, rw_ref, lg_ref, am_ref):
    x = x_ref[...]
    w = w_ref[...]
    g = g_ref[...]
    kf = k_ref[...]

    w_norm = jnp.sqrt(jnp.sum(w * w, axis=0, keepdims=True))
    wn = w / jnp.maximum(w_norm, 1e-12)
    x_norm = jnp.sqrt(jnp.sum(x * x, axis=1, keepdims=True))
    xn = x / jnp.maximum(x_norm, 1e-12)

    logits = jnp.dot(xn, wn, preferred_element_type=jnp.float32) - g
    gated = jnp.maximum(logits, 0.0)
    act_mask = (logits > 0.0).astype(jnp.float32)
    inactive = jnp.max(logits, axis=1, keepdims=True) <= 0.0

    n_experts = logits.shape[1]
    rank = jnp.zeros_like(logits)
    eidx = lax.broadcasted_iota(jnp.int32, (1, n_experts), 1)
    for j in range(n_experts):
        lj = logits[:, j : j + 1]
        ind = jnp.where(
            (lj > logits) | ((lj == logits) & (eidx > j)), 1.0, 0.0
        )
        rank = rank + ind
    fb_mask = (rank < kf).astype(jnp.float32)

    mask = jnp.where(inactive, fb_mask, act_mask)
    neg = jnp.float32(-1e30)
    gated_masked = jnp.where(mask > 0.0, gated, neg)
    m = jnp.max(gated_masked, axis=1, keepdims=True)
    ex = jnp.exp(gated_masked - m)
    rw = ex / jnp.sum(ex, axis=1, keepdims=True)

    rw_ref[...] = rw
    lg_ref[...] = logits
    am_ref[...] = mask


def kernel(hidden_states, sim_matrix, gates, fallback_k):
    b, t, c = hidden_states.shape
    n = b * t
    e = sim_matrix.shape[1]
    flat = hidden_states.reshape(n, c)
    w_flat = sim_matrix.reshape(c * e)
    aux = jnp.concatenate(
        [gates.astype(jnp.float32),
         jnp.full((8,), fallback_k, jnp.float32)]
    )

    mesh = plsc.VectorSubcoreMesh(core_axis_name="c", subcore_axis_name="s")
    out_t = [jax.ShapeDtypeStruct((N_SC * e,), jnp.float32)] * 3
    cp = pltpu.CompilerParams()
    if "needs_layout_passes" in pltpu.CompilerParams.__dataclass_fields__:
        cp = dataclasses.replace(cp, needs_layout_passes=False)
    f = pl.kernel(
        _sc_gating,
        out_type=out_t,
        mesh=mesh,
        compiler_params=cp,
        scratch_types=[
            pltpu.VMEM((2, XBLK, C_DIM), jnp.float32),
            pltpu.VMEM((C_DIM * N_EXP,), jnp.float32),
            pltpu.VMEM((N_EXP, C_DIM), jnp.float32),
            pltpu.VMEM((L,), jnp.float32),
            pltpu.VMEM((XBLK * N_EXP,), jnp.float32),
            pltpu.VMEM((XBLK * N_EXP,), jnp.float32),
            pltpu.VMEM((XBLK * N_EXP,), jnp.float32),
            pltpu.SemaphoreType.DMA((2,)),
        ],
    )
    rw_sc, lg_sc, am_sc = f(flat, w_flat, aux)

    n_tc = n - N_SC
    g2 = gates.reshape(1, e).astype(jnp.float32)
    kvec = jnp.full((1, e), fallback_k, jnp.float32)
    off = N_SC // TC_BLK
    rw_tc, lg_tc, am_tc = pl.pallas_call(
        _tc_gating_block,
        grid=(n_tc // TC_BLK,),
        in_specs=[
            pl.BlockSpec((TC_BLK, c), lambda i: (i + off, 0)),
            pl.BlockSpec((c, e), lambda i: (0, 0)),
            pl.BlockSpec((1, e), lambda i: (0, 0)),
            pl.BlockSpec((1, e), lambda i: (0, 0)),
        ],
        out_specs=[pl.BlockSpec((TC_BLK, e), lambda i: (i, 0))] * 3,
        out_shape=[jax.ShapeDtypeStruct((n_tc, e), jnp.float32)] * 3,
    )(flat, sim_matrix, g2, kvec)

    rw = jnp.concatenate([rw_sc.reshape(N_SC, e), rw_tc], axis=0)
    lg = jnp.concatenate([lg_sc.reshape(N_SC, e), lg_tc], axis=0)
    am = jnp.concatenate([am_sc.reshape(N_SC, e), am_tc], axis=0)
    return rw, lg, am


# speedup vs baseline: 1.0052x; 1.0052x over previous
import dataclasses
import functools

import jax
import jax.numpy as jnp
from jax import lax
from jax.experimental import pallas as pl
from jax.experimental.pallas import tpu as pltpu
from jax.experimental.pallas import tpu_sc as plsc

L = 16
NW = 32
N_TOK = 32768
C_DIM = 768
N_EXP = 8
N_SC = 12288
ROWS_PER_WORKER = N_SC // NW
XBLK = 64
NBLK = ROWS_PER_WORKER // XBLK
NCH = C_DIM // L
R_INFLIGHT = 4
TC_BLK = 2048


def _vgather(v, idx):
    return v.at[idx].get(mode="promise_in_bounds")


def _hsum16(v, iota):
    for sh in (8, 4, 2, 1):
        v = v + _vgather(v, jnp.bitwise_xor(iota, sh))
    return v


def _rsqrt16(x):
    i = lax.bitcast_convert_type(x, jnp.int32)
    i = jnp.int32(0x5F3759DF) - lax.shift_right_logical(i, 1)
    y = lax.bitcast_convert_type(i, jnp.float32)
    for _ in range(3):
        y = y * (jnp.float32(1.5) - jnp.float32(0.5) * x * y * y)
    return y


def _bf16_rne(x):
    u = lax.bitcast_convert_type(x, jnp.uint32)
    half = jnp.uint32(0x7FFF) + (
        lax.shift_right_logical(u, jnp.uint32(16)) & jnp.uint32(1))
    r = (u + half) & jnp.uint32(0xFFFF0000)
    return lax.bitcast_convert_type(r, jnp.float32)


def _sc_gating(x_hbm, w_hbm, aux_hbm, rw_hbm, lg_hbm, am_hbm,
               xbuf, wbuf, wt, auxv, rwst, lgst, amst, xsem):
    cid = lax.axis_index("c")
    sid = lax.axis_index("s")
    wid = sid * 2 + cid
    worker_base = wid * ROWS_PER_WORKER

    iota = lax.iota(jnp.int32, L)

    pltpu.sync_copy(w_hbm, wbuf)
    pltpu.sync_copy(aux_hbm, auxv)

    wsqv = jnp.zeros((L,), jnp.float32)
    for e in range(N_EXP):

        def _wch(i, acc, e=e):
            idx = (i * L + iota) * N_EXP + e
            v = plsc.load_gather(wbuf, [idx])
            return acc + v * v

        acc = lax.fori_loop(0, NCH, _wch, jnp.zeros((L,), jnp.float32))
        wsqv = jnp.where(iota == e, _hsum16(acc, iota), wsqv)

    winv = _rsqrt16(jnp.maximum(wsqv, jnp.float32(1e-24)))
    avec = auxv[pl.ds(0, L)]
    g_bc = [_vgather(avec, jnp.full((L,), e, jnp.int32)) for e in range(N_EXP)]
    k_bc = _vgather(avec, jnp.full((L,), N_EXP, jnp.int32))

    for e in range(N_EXP):
        wibc = _vgather(winv, jnp.full((L,), e, jnp.int32))

        def _wch2(i, _, e=e, wibc=wibc):
            idx = (i * L + iota) * N_EXP + e
            v = plsc.load_gather(wbuf, [idx])
            wt[e, pl.ds(i * L, L)] = _bf16_rne(v * wibc)
            return 0

        lax.fori_loop(0, NCH, _wch2, 0)

    def _start(blk, bb):
        rb = worker_base + blk * XBLK
        pltpu.async_copy(x_hbm.at[pl.ds(rb, XBLK), :], xbuf.at[bb],
                         xsem.at[bb])

    def _wait(bb):
        pltpu.make_async_copy(x_hbm.at[pl.ds(0, XBLK), :], xbuf.at[bb],
                              xsem.at[bb]).wait()

    def _compute(blk, bb):
        rowbase = worker_base + blk * XBLK
        xb = xbuf.at[bb]

        @pl.loop(0, XBLK // L)
        def _group(g):
            row_g = g * L
            z = jnp.zeros((L,), jnp.float32)

            ssv = z
            for q in range(L // R_INFLIGHT):
                row0 = row_g + q * R_INFLIGHT

                def _ss(i, sss, row0=row0):
                    off = i * L
                    out = []
                    for r in range(R_INFLIGHT):
                        xv = xb[row0 + r, pl.ds(off, L)]
                        out.append(sss[r] + xv * xv)
                    return tuple(out)

                sss = lax.fori_loop(0, NCH, _ss,
                                    tuple(z for _ in range(R_INFLIGHT)))
                for r in range(R_INFLIGHT):
                    mcol = iota == (q * R_INFLIGHT + r)
                    ssv = jnp.where(mcol, _hsum16(sss[r], iota), ssv)

            xinv = _rsqrt16(jnp.maximum(ssv, jnp.float32(1e-24)))

            dvs = [z for _ in range(N_EXP)]
            for q in range(L // R_INFLIGHT):
                row0 = row_g + q * R_INFLIGHT
                xibc = [_vgather(xinv,
                                 jnp.full((L,), q * R_INFLIGHT + r, jnp.int32))
                        for r in range(R_INFLIGHT)]

                def _ch(i, accs, row0=row0, xibc=xibc):
                    off = i * L
                    wv = [wt[e, pl.ds(off, L)] for e in range(N_EXP)]
                    naccs = []
                    for r in range(R_INFLIGHT):
                        xv = xb[row0 + r, pl.ds(off, L)]
                        xnr = _bf16_rne(xv * xibc[r])
                        naccs.append(tuple(accs[r][e] + xnr * wv[e]
                                           for e in range(N_EXP)))
                    return tuple(naccs)

                accs0 = tuple(tuple(z for _ in range(N_EXP))
                              for _ in range(R_INFLIGHT))
                accs = lax.fori_loop(0, NCH, _ch, accs0)
                for r in range(R_INFLIGHT):
                    mcol = iota == (q * R_INFLIGHT + r)
                    for e in range(N_EXP):
                        dvs[e] = jnp.where(mcol, _hsum16(accs[r][e], iota),
                                           dvs[e])

            logits = [dvs[e] - g_bc[e] for e in range(N_EXP)]

            act = [logits[e] > jnp.float32(0.0) for e in range(N_EXP)]
            anyact = act[0]
            for e in range(1, N_EXP):
                anyact = jnp.logical_or(anyact, act[e])
            inactive = jnp.logical_not(anyact)

            one = jnp.full((L,), 1.0, jnp.float32)
            zero = jnp.full((L,), 0.0, jnp.float32)
            rank = [zero for _ in range(N_EXP)]
            for a in range(N_EXP):
                for b in range(a + 1, N_EXP):
                    t = jnp.where(logits[a] >= logits[b], one, zero)
                    rank[b] = rank[b] + t
                    rank[a] = rank[a] + (one - t)
            fb = [rank[e] < k_bc for e in range(N_EXP)]

            mask = [jnp.where(inactive, fb[e], act[e]) for e in range(N_EXP)]
            maskf = [jnp.where(mask[e], 1.0, 0.0) for e in range(N_EXP)]
            gated = [jnp.maximum(logits[e], 0.0) for e in range(N_EXP)]
            neg = jnp.float32(-1e30)
            gm = [jnp.where(mask[e], gated[e], neg) for e in range(N_EXP)]
            mx = gm[0]
            for e in range(1, N_EXP):
                mx = jnp.maximum(mx, gm[e])
            ex = [jnp.exp(gm[e] - mx) for e in range(N_EXP)]
            s = ex[0]
            for e in range(1, N_EXP):
                s = s + ex[e]
            rw = [ex[e] / s for e in range(N_EXP)]

            ridx8 = (row_g + iota) * N_EXP
            for e in range(N_EXP):
                idx = ridx8 + e
                plsc.store_scatter(rwst, [idx], rw[e])
                plsc.store_scatter(lgst, [idx], logits[e])
                plsc.store_scatter(amst, [idx], maskf[e])

        outbase = rowbase * N_EXP
        pltpu.sync_copy(rwst, rw_hbm.at[pl.ds(outbase, XBLK * N_EXP)])
        pltpu.sync_copy(lgst, lg_hbm.at[pl.ds(outbase, XBLK * N_EXP)])
        pltpu.sync_copy(amst, am_hbm.at[pl.ds(outbase, XBLK * N_EXP)])

    _start(0, 0)

    @pl.loop(0, NBLK // 2)
    def _pair(p):
        blk0 = p * 2
        _start(blk0 + 1, 1)
        _wait(0)
        _compute(blk0, 0)
        _start(blk0 + 2, 0)
        _wait(1)
        _compute(blk0 + 1, 1)

    _wait(0)


def _tc_gating_block(x_ref, w_ref, g_ref, k_ref, rw_ref, lg_ref, am_ref):
    x = x_ref[...]
    w = w_ref[...]
    g = g_ref[...]
    kf = k_ref[...]

    w_norm = jnp.sqrt(jnp.sum(w * w, axis=0, keepdims=True))
    wn = w / jnp.maximum(w_norm, 1e-12)
    x_norm = jnp.sqrt(jnp.sum(x * x, axis=1, keepdims=True))
    xn = x / jnp.maximum(x_norm, 1e-12)

    logits = jnp.dot(xn, wn, preferred_element_type=jnp.float32) - g
    gated = jnp.maximum(logits, 0.0)
    act_mask = (logits > 0.0).astype(jnp.float32)
    inactive = jnp.max(logits, axis=1, keepdims=True) <= 0.0

    n_experts = logits.shape[1]
    rank = jnp.zeros_like(logits)
    eidx = lax.broadcasted_iota(jnp.int32, (1, n_experts), 1)
    for j in range(n_experts):
        lj = logits[:, j : j + 1]
        ind = jnp.where(
            (lj > logits) | ((lj == logits) & (eidx > j)), 1.0, 0.0
        )
        rank = rank + ind
    fb_mask = (rank < kf).astype(jnp.float32)

    mask = jnp.where(inactive, fb_mask, act_mask)
    neg = jnp.float32(-1e30)
    gated_masked = jnp.where(mask > 0.0, gated, neg)
    m = jnp.max(gated_masked, axis=1, keepdims=True)
    ex = jnp.exp(gated_masked - m)
    rw = ex / jnp.sum(ex, axis=1, keepdims=True)

    rw_ref[...] = rw
    lg_ref[...] = logits
    am_ref[...] = mask


def kernel(hidden_states, sim_matrix, gates, fallback_k):
    b, t, c = hidden_states.shape
    n = b * t
    e = sim_matrix.shape[1]
    flat = hidden_states.reshape(n, c)
    w_flat = sim_matrix.reshape(c * e)
    aux = jnp.concatenate(
        [gates.astype(jnp.float32),
         jnp.full((8,), fallback_k, jnp.float32)]
    )

    mesh = plsc.VectorSubcoreMesh(core_axis_name="c", subcore_axis_name="s")
    out_t = [jax.ShapeDtypeStruct((N_SC * e,), jnp.float32)] * 3
    cp = pltpu.CompilerParams()
    if "needs_layout_passes" in pltpu.CompilerParams.__dataclass_fields__:
        cp = dataclasses.replace(cp, needs_layout_passes=False)
    f = pl.kernel(
        _sc_gating,
        out_type=out_t,
        mesh=mesh,
        compiler_params=cp,
        scratch_types=[
            pltpu.VMEM((2, XBLK, C_DIM), jnp.float32),
            pltpu.VMEM((C_DIM * N_EXP,), jnp.float32),
            pltpu.VMEM((N_EXP, C_DIM), jnp.float32),
            pltpu.VMEM((L,), jnp.float32),
            pltpu.VMEM((XBLK * N_EXP,), jnp.float32),
            pltpu.VMEM((XBLK * N_EXP,), jnp.float32),
            pltpu.VMEM((XBLK * N_EXP,), jnp.float32),
            pltpu.SemaphoreType.DMA((2,)),
        ],
    )
    rw_sc, lg_sc, am_sc = f(flat, w_flat, aux)

    n_tc = n - N_SC
    g2 = gates.reshape(1, e).astype(jnp.float32)
    kvec = jnp.full((1, e), fallback_k, jnp.float32)
    off = N_SC // TC_BLK
    rw_tc, lg_tc, am_tc = pl.pallas_call(
        _tc_gating_block,
        grid=(n_tc // TC_BLK,),
        in_specs=[
            pl.BlockSpec((TC_BLK, c), lambda i: (i + off, 0)),
            pl.BlockSpec((c, e), lambda i: (0, 0)),
            pl.BlockSpec((1, e), lambda i: (0, 0)),
            pl.BlockSpec((1, e), lambda i: (0, 0)),
        ],
        out_specs=[pl.BlockSpec((TC_BLK, e), lambda i: (i, 0))] * 3,
        out_shape=[jax.ShapeDtypeStruct((n_tc, e), jnp.float32)] * 3,
    )(flat, sim_matrix, g2, kvec)

    rw = jnp.concatenate([rw_sc.reshape(N_SC, e), rw_tc], axis=0)
    lg = jnp.concatenate([lg_sc.reshape(N_SC, e), lg_tc], axis=0)
    am = jnp.concatenate([am_sc.reshape(N_SC, e), am_tc], axis=0)
    return rw, lg, am


# speedup vs baseline: 1.0066x; 1.0014x over previous
import dataclasses

import jax
import jax.numpy as jnp
from jax import lax
from jax.experimental import pallas as pl
from jax.experimental.pallas import tpu as pltpu
from jax.experimental.pallas import tpu_sc as plsc

L = 16
NW = 32
N_TOK = 32768
C_DIM = 768
N_EXP = 8
N_SC = 12288
ROWS_PER_WORKER = N_SC // NW
XBLK = 64
NBLK = ROWS_PER_WORKER // XBLK
NCH = C_DIM // L
R_INFLIGHT = 4
TC_BLK = 2048


def _vgather(v, idx):
    return v.at[idx].get(mode="promise_in_bounds")


def _hsum16(v, iota):
    for sh in (8, 4, 2, 1):
        v = v + _vgather(v, jnp.bitwise_xor(iota, sh))
    return v


def _rsqrt16(x):
    i = lax.bitcast_convert_type(x, jnp.int32)
    i = jnp.int32(0x5F3759DF) - lax.shift_right_logical(i, 1)
    y = lax.bitcast_convert_type(i, jnp.float32)
    for _ in range(3):
        y = y * (jnp.float32(1.5) - jnp.float32(0.5) * x * y * y)
    return y


def _bf16_rne(x):
    u = lax.bitcast_convert_type(x, jnp.uint32)
    half = jnp.uint32(0x7FFF) + (
        lax.shift_right_logical(u, jnp.uint32(16)) & jnp.uint32(1))
    r = (u + half) & jnp.uint32(0xFFFF0000)
    return lax.bitcast_convert_type(r, jnp.float32)


def _sc_gating(x_hbm, w_hbm, aux_hbm, rw_hbm, lg_hbm, am_hbm,
               xbuf, wbuf, wt, auxv, rwst, lgst, amst, xsem):
    cid = lax.axis_index("c")
    sid = lax.axis_index("s")
    wid = sid * 2 + cid
    worker_base = wid * ROWS_PER_WORKER

    iota = lax.iota(jnp.int32, L)

    pltpu.sync_copy(w_hbm, wbuf)
    pltpu.sync_copy(aux_hbm, auxv)

    wsqv = jnp.zeros((L,), jnp.float32)
    for e in range(N_EXP):

        def _wch(i, acc, e=e):
            idx = (i * L + iota) * N_EXP + e
            v = plsc.load_gather(wbuf, [idx])
            return acc + v * v

        acc = lax.fori_loop(0, NCH, _wch, jnp.zeros((L,), jnp.float32))
        wsqv = jnp.where(iota == e, _hsum16(acc, iota), wsqv)

    winv = _rsqrt16(jnp.maximum(wsqv, jnp.float32(1e-24)))
    avec = auxv[pl.ds(0, L)]
    g_bc = [_vgather(avec, jnp.full((L,), e, jnp.int32)) for e in range(N_EXP)]
    k_bc = _vgather(avec, jnp.full((L,), N_EXP, jnp.int32))

    for e in range(N_EXP):
        wibc = _vgather(winv, jnp.full((L,), e, jnp.int32))

        def _wch2(i, _, e=e, wibc=wibc):
            idx = (i * L + iota) * N_EXP + e
            v = plsc.load_gather(wbuf, [idx])
            wt[e, pl.ds(i * L, L)] = _bf16_rne(v * wibc)
            return 0

        lax.fori_loop(0, NCH, _wch2, 0)

    def _start(blk, bb):
        rb = worker_base + blk * XBLK
        pltpu.async_copy(x_hbm.at[pl.ds(rb, XBLK), :], xbuf.at[bb],
                         xsem.at[bb])

    def _wait(bb):
        pltpu.make_async_copy(x_hbm.at[pl.ds(0, XBLK), :], xbuf.at[bb],
                              xsem.at[bb]).wait()

    def _compute(blk, bb):
        rowbase = worker_base + blk * XBLK
        xb = xbuf.at[bb]

        @pl.loop(0, XBLK // L)
        def _group(g):
            row_g = g * L
            z = jnp.zeros((L,), jnp.float32)

            ssv = z
            for q in range(L // R_INFLIGHT):
                row0 = row_g + q * R_INFLIGHT

                def _ss(i, sss, row0=row0):
                    off = i * L
                    out = []
                    for r in range(R_INFLIGHT):
                        xv = xb[row0 + r, pl.ds(off, L)]
                        out.append(sss[r] + xv * xv)
                    return tuple(out)

                sss = lax.fori_loop(0, NCH, _ss,
                                    tuple(z for _ in range(R_INFLIGHT)))
                for r in range(R_INFLIGHT):
                    mcol = iota == (q * R_INFLIGHT + r)
                    ssv = jnp.where(mcol, _hsum16(sss[r], iota), ssv)

            xinv = _rsqrt16(jnp.maximum(ssv, jnp.float32(1e-24)))

            dvs = [z for _ in range(N_EXP)]
            for q in range(L // R_INFLIGHT):
                row0 = row_g + q * R_INFLIGHT
                xibc = [_vgather(xinv,
                                 jnp.full((L,), q * R_INFLIGHT + r, jnp.int32))
                        for r in range(R_INFLIGHT)]

                def _ch(i, accs, row0=row0, xibc=xibc):
                    off = i * L
                    wv = [wt[e, pl.ds(off, L)] for e in range(N_EXP)]
                    naccs = []
                    for r in range(R_INFLIGHT):
                        xv = xb[row0 + r, pl.ds(off, L)]
                        xnr = _bf16_rne(xv * xibc[r])
                        naccs.append(tuple(accs[r][e] + xnr * wv[e]
                                           for e in range(N_EXP)))
                    return tuple(naccs)

                accs0 = tuple(tuple(z for _ in range(N_EXP))
                              for _ in range(R_INFLIGHT))
                accs = lax.fori_loop(0, NCH, _ch, accs0)
                for r in range(R_INFLIGHT):
                    mcol = iota == (q * R_INFLIGHT + r)
                    for e in range(N_EXP):
                        dvs[e] = jnp.where(mcol, _hsum16(accs[r][e], iota),
                                           dvs[e])

            logits = [dvs[e] - g_bc[e] for e in range(N_EXP)]

            act = [logits[e] > jnp.float32(0.0) for e in range(N_EXP)]
            anyact = act[0]
            for e in range(1, N_EXP):
                anyact = jnp.logical_or(anyact, act[e])
            inactive = jnp.logical_not(anyact)

            one = jnp.full((L,), 1.0, jnp.float32)
            zero = jnp.full((L,), 0.0, jnp.float32)
            rank = [zero for _ in range(N_EXP)]
            for a in range(N_EXP):
                for b in range(a + 1, N_EXP):
                    t = jnp.where(logits[a] >= logits[b], one, zero)
                    rank[b] = rank[b] + t
                    rank[a] = rank[a] + (one - t)
            fb = [rank[e] < k_bc for e in range(N_EXP)]

            mask = [jnp.where(inactive, fb[e], act[e]) for e in range(N_EXP)]
            maskf = [jnp.where(mask[e], 1.0, 0.0) for e in range(N_EXP)]
            gated = [jnp.maximum(logits[e], 0.0) for e in range(N_EXP)]
            neg = jnp.float32(-1e30)
            gm = [jnp.where(mask[e], gated[e], neg) for e in range(N_EXP)]
            mx = gm[0]
            for e in range(1, N_EXP):
                mx = jnp.maximum(mx, gm[e])
            ex = [jnp.exp(gm[e] - mx) for e in range(N_EXP)]
            s = ex[0]
            for e in range(1, N_EXP):
                s = s + ex[e]
            rw = [ex[e] / s for e in range(N_EXP)]

            ridx8 = (row_g + iota) * N_EXP
            for e in range(N_EXP):
                idx = ridx8 + e
                plsc.store_scatter(rwst, [idx], rw[e])
                plsc.store_scatter(lgst, [idx], logits[e])
                plsc.store_scatter(amst, [idx], maskf[e])

        outbase = rowbase * N_EXP
        pltpu.sync_copy(rwst, rw_hbm.at[pl.ds(outbase, XBLK * N_EXP)])
        pltpu.sync_copy(lgst, lg_hbm.at[pl.ds(outbase, XBLK * N_EXP)])
        pltpu.sync_copy(amst, am_hbm.at[pl.ds(outbase, XBLK * N_EXP)])

    _start(0, 0)

    @pl.loop(0, NBLK // 2)
    def _pair(p):
        blk0 = p * 2
        _start(blk0 + 1, 1)
        _wait(0)
        _compute(blk0, 0)
        _start(blk0 + 2, 0)
        _wait(1)
        _compute(blk0 + 1, 1)

    _wait(0)


def _tc_gating_block(x_ref, w_ref, g_ref, k_ref, rw_ref, lg_ref, am_ref):
    x = x_ref[...]
    w = w_ref[...]
    g = g_ref[...]
    kf = k_ref[...]

    w_norm = jnp.sqrt(jnp.sum(w * w, axis=0, keepdims=True))
    wn = w / jnp.maximum(w_norm, 1e-12)
    x_norm = jnp.sqrt(jnp.sum(x * x, axis=1, keepdims=True))
    xn = x / jnp.maximum(x_norm, 1e-12)

    logits = jnp.dot(xn, wn, preferred_element_type=jnp.float32) - g
    gated = jnp.maximum(logits, 0.0)
    act_mask = (logits > 0.0).astype(jnp.float32)
    inactive = jnp.max(logits, axis=1, keepdims=True) <= 0.0

    n_experts = logits.shape[1]
    rank = jnp.zeros_like(logits)
    eidx = lax.broadcasted_iota(jnp.int32, (1, n_experts), 1)
    for j in range(n_experts):
        lj = logits[:, j : j + 1]
        ind = jnp.where(
            (lj > logits) | ((lj == logits) & (eidx > j)), 1.0, 0.0
        )
        rank = rank + ind
    fb_mask = (rank < kf).astype(jnp.float32)

    mask = jnp.where(inactive, fb_mask, act_mask)
    neg = jnp.float32(-1e30)
    gated_masked = jnp.where(mask > 0.0, gated, neg)
    m = jnp.max(gated_masked, axis=1, keepdims=True)
    ex = jnp.exp(gated_masked - m)
    rw = ex / jnp.sum(ex, axis=1, keepdims=True)

    rw_ref[...] = rw
    lg_ref[...] = logits
    am_ref[...] = mask


def kernel(hidden_states, sim_matrix, gates, fallback_k):
    b, t, c = hidden_states.shape
    n = b * t
    e = sim_matrix.shape[1]
    flat = hidden_states.reshape(n, c)
    w_flat = sim_matrix.reshape(c * e)
    aux = jnp.concatenate(
        [gates.astype(jnp.float32),
         jnp.full((8,), fallback_k, jnp.float32)]
    )

    mesh = plsc.VectorSubcoreMesh(core_axis_name="c", subcore_axis_name="s")
    out_t = [jax.ShapeDtypeStruct((N_SC * e,), jnp.float32)] * 3
    cp = pltpu.CompilerParams()
    if "needs_layout_passes" in pltpu.CompilerParams.__dataclass_fields__:
        cp = dataclasses.replace(cp, needs_layout_passes=False)
    f = pl.kernel(
        _sc_gating,
        out_type=out_t,
        mesh=mesh,
        compiler_params=cp,
        scratch_types=[
            pltpu.VMEM((2, XBLK, C_DIM), jnp.float32),
            pltpu.VMEM((C_DIM * N_EXP,), jnp.float32),
            pltpu.VMEM((N_EXP, C_DIM), jnp.float32),
            pltpu.VMEM((L,), jnp.float32),
            pltpu.VMEM((XBLK * N_EXP,), jnp.float32),
            pltpu.VMEM((XBLK * N_EXP,), jnp.float32),
            pltpu.VMEM((XBLK * N_EXP,), jnp.float32),
            pltpu.SemaphoreType.DMA((2,)),
        ],
    )
    rw_sc, lg_sc, am_sc = f(flat, w_flat, aux)

    n_tc = n - N_SC
    g2 = gates.reshape(1, e).astype(jnp.float32)
    kvec = jnp.full((1, e), fallback_k, jnp.float32)
    off = N_SC // TC_BLK
    rw_tc, lg_tc, am_tc = pl.pallas_call(
        _tc_gating_block,
        grid=(n_tc // TC_BLK,),
        in_specs=[
            pl.BlockSpec((TC_BLK, c), lambda i: (i + off, 0)),
            pl.BlockSpec((c, e), lambda i: (0, 0)),
            pl.BlockSpec((1, e), lambda i: (0, 0)),
            pl.BlockSpec((1, e), lambda i: (0, 0)),
        ],
        out_specs=[pl.BlockSpec((TC_BLK, e), lambda i: (i, 0))] * 3,
        out_shape=[jax.ShapeDtypeStruct((n_tc, e), jnp.float32)] * 3,
    )(flat, sim_matrix, g2, kvec)

    rw = jnp.concatenate([rw_sc.reshape(N_SC, e), rw_tc], axis=0)
    lg = jnp.concatenate([lg_sc.reshape(N_SC, e), lg_tc], axis=0)
    am = jnp.concatenate([am_sc.reshape(N_SC, e), am_tc], axis=0)
    return rw, lg, am
